# Initial kernel scaffold; baseline (speedup 1.0000x reference)
import jax
import jax.numpy as jnp
from jax.experimental import pallas as pl


def kernel(inputs, embedding_weight):
    raise NotImplementedError("write your pallas kernel here")


# speedup vs baseline: 1.3907x; 1.3907x over previous
import functools

import jax
import jax.numpy as jnp
from jax import lax
from jax.experimental import pallas as pl
from jax.experimental.pallas import tpu as pltpu
from jax.experimental.pallas import tpu_sc as plsc

N_TOKENS = 16384
N_CODES = 8192
DIM = 32
COMMIT = 0.25

BT = 512
BC = 2048
NTB = N_TOKENS // BT
NCB = N_CODES // BC

SC_NC = 2
SC_NS = 16
SC_NW = SC_NC * SC_NS
ROWS_W = N_TOKENS // SC_NW
IDX_CH = 128
N_CH = ROWS_W // IDX_CH


def _argmin_body(x_ref, e_ref, xn_ref, en_ref, idx_out,
                 hmin_ref, hidx_ref, accv_ref, acci_ref):
    j = pl.program_id(1)
    mm = lax.dot_general(
        x_ref[...], e_ref[...],
        dimension_numbers=(((1,), (1,)), ((), ())),
        preferred_element_type=jnp.float32,
    )
    d = (xn_ref[...] + en_ref[0:1, :]) - 2.0 * mm
    lmin = jnp.min(d, axis=1, keepdims=True)
    iota = lax.broadcasted_iota(jnp.int32, d.shape, 1)
    larg = jnp.min(
        jnp.where(d == lmin, iota, jnp.int32(2**30)), axis=1, keepdims=True
    ) + j * BC

    @pl.when((j == 0) | (j == 2))
    def _():
        hmin_ref[...] = lmin
        hidx_ref[...] = larg

    @pl.when((j == 1) | (j == 3))
    def _():
        better = lmin < hmin_ref[...]
        hmin_ref[...] = jnp.where(better, lmin, hmin_ref[...])
        hidx_ref[...] = jnp.where(better, larg, hidx_ref[...])

    @pl.when(j == 1)
    def _():
        accv_ref[...] = hmin_ref[...].astype(jnp.bfloat16).astype(jnp.float32)
        acci_ref[...] = hidx_ref[...]

    @pl.when(j == NCB - 1)
    def _():
        win = hmin_ref[...] < accv_ref[...]
        idx_out[...] = jnp.where(win, hidx_ref[...], acci_ref[...])


def _sc_gather_hist_body(idx_hbm, emb_hbm, q_hbm, hist_hbm,
                         idx_v, rows_v, hist_v, sem):
    wid = lax.axis_index("s") * SC_NC + lax.axis_index("c")
    base = wid * ROWS_W

    for k in range(N_CH):
        pltpu.sync_copy(idx_hbm.at[pl.ds(base + k * IDX_CH, IDX_CH)],
                        idx_v.at[k])

    copies = [
        pltpu.async_copy(emb_hbm.at[idx_v.at[k]],
                         rows_v.at[pl.ds(k * IDX_CH, IDX_CH)], sem)
        for k in range(N_CH)
    ]

    def _zero(t, _):
        hist_v[pl.ds(t * 16, 16)] = jnp.zeros((16,), jnp.float32)
        return _
    lax.fori_loop(0, N_CODES // 16, _zero, None)

    ones = jnp.ones((16,), jnp.float32)
    lane_iota = lax.broadcasted_iota(jnp.int32, (16,), 0)
    for k in range(N_CH):
        for l in range(IDX_CH // 16):
            iv = idx_v[k, pl.ds(l * 16, 16)]
            for lane in range(16):
                plsc.addupdate_scatter(hist_v, [iv], ones,
                                       mask=lane_iota == lane)

    for c in copies:
        c.wait()

    pltpu.sync_copy(rows_v, q_hbm.at[pl.ds(base, ROWS_W)])
    pltpu.sync_copy(hist_v, hist_hbm.at[wid])


def _finalize_body(x_ref, q_ref, h_ref, qst_ref, loss_ref, perp_ref):
    x = x_ref[...]
    q = q_ref[...]
    dq = q - x
    qst_ref[...] = x + dq
    loss = (1.0 + COMMIT) * jnp.sum(dq * dq) / (N_TOKENS * DIM)
    loss_ref[...] = loss.reshape(1, 1)
    counts = jnp.sum(h_ref[...], axis=0)
    avg = counts * (1.0 / N_TOKENS)
    ent = -jnp.sum(avg * jnp.log(avg + 1e-10))
    perp_ref[...] = jnp.exp(ent).reshape(1, 1)


def kernel(inputs, embedding_weight):
    x = inputs
    emb = embedding_weight
    xn = jnp.sum(x ** 2, axis=1, keepdims=True)
    en = jnp.sum(emb ** 2, axis=1)
    en8 = jnp.broadcast_to(en.reshape(1, N_CODES), (8, N_CODES))

    idx2 = pl.pallas_call(
        _argmin_body,
        grid=(NTB, NCB),
        in_specs=[
            pl.BlockSpec((BT, DIM), lambda i, j: (i, 0)),
            pl.BlockSpec((BC, DIM), lambda i, j: (j, 0)),
            pl.BlockSpec((BT, 1), lambda i, j: (i, 0)),
            pl.BlockSpec((8, BC), lambda i, j: (0, j)),
        ],
        out_specs=pl.BlockSpec((BT, 1), lambda i, j: (i, 0)),
        out_shape=jax.ShapeDtypeStruct((N_TOKENS, 1), jnp.int32),
        scratch_shapes=[
            pltpu.VMEM((BT, 1), jnp.float32),
            pltpu.VMEM((BT, 1), jnp.int32),
            pltpu.VMEM((BT, 1), jnp.float32),
            pltpu.VMEM((BT, 1), jnp.int32),
        ],
    )(x, emb, xn, en8)
    idx = idx2.reshape(N_TOKENS)

    sc_call = pl.kernel(
        _sc_gather_hist_body,
        out_type=(
            jax.ShapeDtypeStruct((N_TOKENS, DIM), jnp.float32),
            jax.ShapeDtypeStruct((SC_NW, N_CODES), jnp.float32),
        ),
        mesh=plsc.VectorSubcoreMesh(core_axis_name="c", subcore_axis_name="s"),
        scratch_types=[
            pltpu.VMEM((N_CH, IDX_CH), jnp.int32),
            pltpu.VMEM((ROWS_W, DIM), jnp.float32),
            pltpu.VMEM((N_CODES,), jnp.float32),
            pltpu.SemaphoreType.DMA,
        ],
        compiler_params=pltpu.CompilerParams(
            needs_layout_passes=False, use_tc_tiling_on_sc=False),
    )
    quant, hist = sc_call(idx, emb)

    qst, loss, perp = pl.pallas_call(
        _finalize_body,
        out_shape=(
            jax.ShapeDtypeStruct((N_TOKENS, DIM), jnp.float32),
            jax.ShapeDtypeStruct((1, 1), jnp.float32),
            jax.ShapeDtypeStruct((1, 1), jnp.float32),
        ),
    )(x, quant, hist)

    return qst, loss.reshape(()), perp.reshape(())
